# baseline (speedup 1.0000x reference)
import jax
import jax.numpy as jnp
from jax.experimental import pallas as pl

SIGMA = 0.08
H = 224
W = 224
N = H * W
C = 128
R = 3584
NT = N // R
HALO = 904
P = R + 2 * HALO
NM = 9


def _sh2(a, dx, dy):
    if dx == 1:
        a = jnp.concatenate([a[1:, :], jnp.zeros((1, W), a.dtype)], axis=0)
    elif dx == -1:
        a = jnp.concatenate([jnp.zeros((1, W), a.dtype), a[:-1, :]], axis=0)
    if dy == 1:
        a = jnp.concatenate([a[:, 1:], jnp.zeros((H, 1), a.dtype)], axis=1)
    elif dy == -1:
        a = jnp.concatenate([jnp.zeros((H, 1), a.dtype), a[:, :-1]], axis=1)
    return a


def _maps_kernel(r_ref, o_ref):
    r = r_ref[...]
    rmin = jnp.min(r)
    rmax = jnp.max(r)
    r01 = (r - rmin) / jnp.maximum(rmax - rmin, 1e-12)
    inv = 1.0 / (2.0 * SIGMA * SIGMA + 1e-12)
    dv = r01[1:, :] - r01[:-1, :]
    wv = jnp.exp(-(dv * dv) * inv)
    zrow = jnp.zeros((1, W), jnp.float32)
    wdn = jnp.concatenate([wv, zrow], axis=0)
    wup = jnp.concatenate([zrow, wv], axis=0)
    dh = r01[:, 1:] - r01[:, :-1]
    wh = jnp.exp(-(dh * dh) * inv)
    zcol = jnp.zeros((H, 1), jnp.float32)
    wrt = jnp.concatenate([wh, zcol], axis=1)
    wlf = jnp.concatenate([zcol, wh], axis=1)
    deg = wup + wdn + wlf + wrt
    s = (deg + 1e-12) ** -0.5
    s2 = (deg + 1.0) ** -0.5
    o_ref[0] = wup * s * _sh2(s, -1, 0)
    o_ref[1] = wdn * s * _sh2(s, 1, 0)
    o_ref[2] = wlf * s * _sh2(s, 0, -1)
    o_ref[3] = wrt * s * _sh2(s, 0, 1)
    o_ref[4] = wup * s2 * _sh2(s2, -1, 0)
    o_ref[5] = wdn * s2 * _sh2(s2, 1, 0)
    o_ref[6] = wlf * s2 * _sh2(s2, 0, -1)
    o_ref[7] = wrt * s2 * _sh2(s2, 0, 1)
    o_ref[8] = s2 * s2


def _aggs(q, qoff, m, pa, sz, c0):
    o = pa - qoff
    wu = m[pa:pa + sz, c0:c0 + 1]
    wd = m[pa:pa + sz, c0 + 1:c0 + 2]
    wl = m[pa:pa + sz, c0 + 2:c0 + 3]
    wr = m[pa:pa + sz, c0 + 3:c0 + 4]
    return (wu * q[o - W:o - W + sz] + wd * q[o + W:o + W + sz]
            + wl * q[o - 1:o - 1 + sz] + wr * q[o + 1:o + 1 + sz])


def _main_kernel(xp_ref, xc_ref, xn_ref, mp_ref, mc_ref, mn_ref,
                 w1_ref, b1_ref, w2_ref, b2_ref, gam_ref, bet_ref, gate_ref,
                 o_ref):
    x3 = jnp.concatenate([xp_ref[...], xc_ref[...], xn_ref[...]], axis=0)
    m3 = jnp.concatenate([mp_ref[...], mc_ref[...], mn_ref[...]], axis=0)
    xpad = x3[R - HALO:2 * R + HALO]
    m = m3[R - HALO:2 * R + HALO]

    d1 = _aggs(xpad, 0, m, 224, P - 448, 0)
    sz1 = R + 912
    d2 = _aggs(d1, 224, m, 448, sz1, 0)

    w1 = w1_ref[...]
    dot = lambda u, v: jnp.dot(u, v, preferred_element_type=jnp.float32)
    z1 = (dot(xpad[448:448 + sz1], w1[0:C])
          + dot(d1[224:224 + sz1], w1[C:2 * C])
          + dot(d2, w1[2 * C:3 * C]))

    sz2 = R + 464
    y1 = (_aggs(z1, 448, m, 672, sz2, 4)
          + m[672:672 + sz2, 8:9] * z1[224:224 + sz2] + b1_ref[...])

    mu = jnp.mean(y1, axis=-1, keepdims=True)
    var = jnp.mean((y1 - mu) ** 2, axis=-1, keepdims=True)
    y = (y1 - mu) * jax.lax.rsqrt(var + 1e-5) * gam_ref[...] + bet_ref[...]
    y = jnp.maximum(y, 0.0)

    z2 = dot(y, w2_ref[...])
    y2 = (_aggs(z2, 672, m, HALO, R, 4)
          + m[HALO:HALO + R, 8:9] * z2[232:232 + R] + b2_ref[...])

    xc = xc_ref[...]
    g = jnp.clip(gate_ref[0, 0], 0.0, 1.0)
    o_ref[...] = jnp.transpose(xc + g * (y2 - xc), (1, 0))


def kernel(feat, rgt_map, W1, b1, W2, b2, gamma, beta, gate):
    r2d = rgt_map.reshape(H, W)
    maps3 = pl.pallas_call(
        _maps_kernel,
        out_shape=jax.ShapeDtypeStruct((NM, H, W), jnp.float32),
    )(r2d)
    maps = jnp.transpose(maps3.reshape(NM, N), (1, 0))

    x2d = jnp.transpose(feat, (0, 2, 3, 1)).reshape(N, C)

    xspec = lambda f: pl.BlockSpec((R, C), lambda i, f=f: (f(i), 0))
    mspec = lambda f: pl.BlockSpec((R, NM), lambda i, f=f: (f(i), 0))
    prev = lambda i: jnp.maximum(i - 1, 0)
    nxt = lambda i: jnp.minimum(i + 1, NT - 1)
    cur = lambda i: i
    full = lambda r, c: pl.BlockSpec((r, c), lambda i: (0, 0))

    out_rows = pl.pallas_call(
        _main_kernel,
        grid=(NT,),
        in_specs=[
            xspec(prev), xspec(cur), xspec(nxt),
            mspec(prev), mspec(cur), mspec(nxt),
            full(3 * C, C), full(1, C), full(C, C), full(1, C),
            full(1, C), full(1, C), full(1, 1),
        ],
        out_specs=pl.BlockSpec((C, R), lambda i: (0, i)),
        out_shape=jax.ShapeDtypeStruct((C, N), jnp.float32),
    )(x2d, x2d, x2d, maps, maps, maps,
      W1, b1.reshape(1, C), W2, b2.reshape(1, C),
      gamma.reshape(1, C), beta.reshape(1, C),
      jnp.asarray(gate, jnp.float32).reshape(1, 1))

    return out_rows.reshape(1, C, H, W)


# speedup vs baseline: 1.1583x; 1.1583x over previous
import jax
import jax.numpy as jnp
from jax.experimental import pallas as pl

SIGMA = 0.08
H = 224
W = 224
N = H * W
C = 128
R = 3584
NT = N // R
HALO = 904
P = R + 2 * HALO
NM = 9


def _sh2(a, dx, dy):
    if dx == 1:
        a = jnp.concatenate([a[1:, :], jnp.zeros((1, W), a.dtype)], axis=0)
    elif dx == -1:
        a = jnp.concatenate([jnp.zeros((1, W), a.dtype), a[:-1, :]], axis=0)
    if dy == 1:
        a = jnp.concatenate([a[:, 1:], jnp.zeros((H, 1), a.dtype)], axis=1)
    elif dy == -1:
        a = jnp.concatenate([jnp.zeros((H, 1), a.dtype), a[:, :-1]], axis=1)
    return a


def _maps_kernel(r_ref, o_ref):
    r = r_ref[...]
    rmin = jnp.min(r)
    rmax = jnp.max(r)
    r01 = (r - rmin) / jnp.maximum(rmax - rmin, 1e-12)
    inv = 1.0 / (2.0 * SIGMA * SIGMA + 1e-12)
    dv = r01[1:, :] - r01[:-1, :]
    wv = jnp.exp(-(dv * dv) * inv)
    zrow = jnp.zeros((1, W), jnp.float32)
    wdn = jnp.concatenate([wv, zrow], axis=0)
    wup = jnp.concatenate([zrow, wv], axis=0)
    dh = r01[:, 1:] - r01[:, :-1]
    wh = jnp.exp(-(dh * dh) * inv)
    zcol = jnp.zeros((H, 1), jnp.float32)
    wrt = jnp.concatenate([wh, zcol], axis=1)
    wlf = jnp.concatenate([zcol, wh], axis=1)
    deg = wup + wdn + wlf + wrt
    s = (deg + 1e-12) ** -0.5
    s2 = (deg + 1.0) ** -0.5
    o_ref[0] = wup * s * _sh2(s, -1, 0)
    o_ref[1] = wdn * s * _sh2(s, 1, 0)
    o_ref[2] = wlf * s * _sh2(s, 0, -1)
    o_ref[3] = wrt * s * _sh2(s, 0, 1)
    o_ref[4] = wup * s2 * _sh2(s2, -1, 0)
    o_ref[5] = wdn * s2 * _sh2(s2, 1, 0)
    o_ref[6] = wlf * s2 * _sh2(s2, 0, -1)
    o_ref[7] = wrt * s2 * _sh2(s2, 0, 1)
    o_ref[8] = s2 * s2


def _aggs(q, qoff, m, pa, sz, c0):
    o = pa - qoff
    wu = m[pa:pa + sz, c0:c0 + 1]
    wd = m[pa:pa + sz, c0 + 1:c0 + 2]
    wl = m[pa:pa + sz, c0 + 2:c0 + 3]
    wr = m[pa:pa + sz, c0 + 3:c0 + 4]
    return (wu * q[o - W:o - W + sz] + wd * q[o + W:o + W + sz]
            + wl * q[o - 1:o - 1 + sz] + wr * q[o + 1:o + 1 + sz])


def _main_kernel(xp_ref, xc_ref, xn_ref, mp_ref, mc_ref, mn_ref,
                 w1_ref, b1_ref, w2_ref, b2_ref, gam_ref, bet_ref, gate_ref,
                 o_ref):
    x3 = jnp.concatenate([xp_ref[...], xc_ref[...], xn_ref[...]], axis=0)
    m3 = jnp.concatenate([mp_ref[...], mc_ref[...], mn_ref[...]], axis=0)
    xpad = x3[R - HALO:2 * R + HALO]
    m = m3[R - HALO:2 * R + HALO]

    d1 = _aggs(xpad, 0, m, 224, P - 448, 0)
    sz1 = R + 912
    d2 = _aggs(d1, 224, m, 448, sz1, 0)

    w1 = w1_ref[...]
    dot = lambda u, v: jnp.dot(u, v, preferred_element_type=jnp.float32)
    z1 = (dot(xpad[448:448 + sz1], w1[0:C])
          + dot(d1[224:224 + sz1], w1[C:2 * C])
          + dot(d2, w1[2 * C:3 * C]))

    sz2 = R + 464
    y1 = (_aggs(z1, 448, m, 672, sz2, 4)
          + m[672:672 + sz2, 8:9] * z1[224:224 + sz2] + b1_ref[...])

    mu = jnp.mean(y1, axis=-1, keepdims=True)
    var = jnp.mean((y1 - mu) ** 2, axis=-1, keepdims=True)
    y = (y1 - mu) * jax.lax.rsqrt(var + 1e-5) * gam_ref[...] + bet_ref[...]
    y = jnp.maximum(y, 0.0)

    z2 = dot(y, w2_ref[...])
    y2 = (_aggs(z2, 672, m, HALO, R, 4)
          + m[HALO:HALO + R, 8:9] * z2[232:232 + R] + b2_ref[...])

    xc = xc_ref[...]
    g = jnp.clip(gate_ref[0, 0], 0.0, 1.0)
    o_ref[...] = xc + g * (y2 - xc)


def kernel(feat, rgt_map, W1, b1, W2, b2, gamma, beta, gate):
    r2d = rgt_map.reshape(H, W)
    maps3 = pl.pallas_call(
        _maps_kernel,
        out_shape=jax.ShapeDtypeStruct((NM, H, W), jnp.float32),
    )(r2d)
    maps = jnp.transpose(maps3.reshape(NM, N), (1, 0))

    x2d = jnp.transpose(feat, (0, 2, 3, 1)).reshape(N, C)

    xspec = lambda f: pl.BlockSpec((R, C), lambda i, f=f: (f(i), 0))
    mspec = lambda f: pl.BlockSpec((R, NM), lambda i, f=f: (f(i), 0))
    prev = lambda i: jnp.maximum(i - 1, 0)
    nxt = lambda i: jnp.minimum(i + 1, NT - 1)
    cur = lambda i: i
    full = lambda r, c: pl.BlockSpec((r, c), lambda i: (0, 0))

    out_rows = pl.pallas_call(
        _main_kernel,
        grid=(NT,),
        in_specs=[
            xspec(prev), xspec(cur), xspec(nxt),
            mspec(prev), mspec(cur), mspec(nxt),
            full(3 * C, C), full(1, C), full(C, C), full(1, C),
            full(1, C), full(1, C), full(1, 1),
        ],
        out_specs=pl.BlockSpec((R, C), lambda i: (i, 0)),
        out_shape=jax.ShapeDtypeStruct((N, C), jnp.float32),
    )(x2d, x2d, x2d, maps, maps, maps,
      W1, b1.reshape(1, C), W2, b2.reshape(1, C),
      gamma.reshape(1, C), beta.reshape(1, C),
      jnp.asarray(gate, jnp.float32).reshape(1, 1))

    return jnp.transpose(out_rows.reshape(1, H, W, C), (0, 3, 1, 2))


# speedup vs baseline: 1.1607x; 1.0020x over previous
import jax
import jax.numpy as jnp
from jax.experimental import pallas as pl

SIGMA = 0.08
H = 224
W = 224
N = H * W
C = 128
R = 3584
NT = N // R
HALO = 904
P = R + 2 * HALO
NM = 9


def _sh2(a, dx, dy):
    if dx == 1:
        a = jnp.concatenate([a[1:, :], jnp.zeros((1, W), a.dtype)], axis=0)
    elif dx == -1:
        a = jnp.concatenate([jnp.zeros((1, W), a.dtype), a[:-1, :]], axis=0)
    if dy == 1:
        a = jnp.concatenate([a[:, 1:], jnp.zeros((H, 1), a.dtype)], axis=1)
    elif dy == -1:
        a = jnp.concatenate([jnp.zeros((H, 1), a.dtype), a[:, :-1]], axis=1)
    return a


def _maps_kernel(r_ref, o_ref):
    r = r_ref[...]
    rmin = jnp.min(r)
    rmax = jnp.max(r)
    r01 = (r - rmin) / jnp.maximum(rmax - rmin, 1e-12)
    inv = 1.0 / (2.0 * SIGMA * SIGMA + 1e-12)
    dv = r01[1:, :] - r01[:-1, :]
    wv = jnp.exp(-(dv * dv) * inv)
    zrow = jnp.zeros((1, W), jnp.float32)
    wdn = jnp.concatenate([wv, zrow], axis=0)
    wup = jnp.concatenate([zrow, wv], axis=0)
    dh = r01[:, 1:] - r01[:, :-1]
    wh = jnp.exp(-(dh * dh) * inv)
    zcol = jnp.zeros((H, 1), jnp.float32)
    wrt = jnp.concatenate([wh, zcol], axis=1)
    wlf = jnp.concatenate([zcol, wh], axis=1)
    deg = wup + wdn + wlf + wrt
    s = (deg + 1e-12) ** -0.5
    s2 = (deg + 1.0) ** -0.5
    o_ref[0] = wup * s * _sh2(s, -1, 0)
    o_ref[1] = wdn * s * _sh2(s, 1, 0)
    o_ref[2] = wlf * s * _sh2(s, 0, -1)
    o_ref[3] = wrt * s * _sh2(s, 0, 1)
    o_ref[4] = wup * s2 * _sh2(s2, -1, 0)
    o_ref[5] = wdn * s2 * _sh2(s2, 1, 0)
    o_ref[6] = wlf * s2 * _sh2(s2, 0, -1)
    o_ref[7] = wrt * s2 * _sh2(s2, 0, 1)
    o_ref[8] = s2 * s2


def _aggs(q, qoff, m, pa, sz, c0):
    o = pa - qoff
    wu = m[pa:pa + sz, c0:c0 + 1]
    wd = m[pa:pa + sz, c0 + 1:c0 + 2]
    wl = m[pa:pa + sz, c0 + 2:c0 + 3]
    wr = m[pa:pa + sz, c0 + 3:c0 + 4]
    return (wu * q[o - W:o - W + sz] + wd * q[o + W:o + W + sz]
            + wl * q[o - 1:o - 1 + sz] + wr * q[o + 1:o + 1 + sz])


def _main_kernel(xp_ref, xc_ref, xn_ref, mp_ref, mc_ref, mn_ref,
                 w1_ref, b1_ref, w2_ref, b2_ref, gam_ref, bet_ref, gate_ref,
                 o_ref):
    xpad = jnp.concatenate(
        [xp_ref[R - HALO:, :], xc_ref[...], xn_ref[:HALO, :]], axis=0)
    m = jnp.concatenate(
        [mp_ref[R - HALO:, :], mc_ref[...], mn_ref[:HALO, :]], axis=0)

    d1 = _aggs(xpad, 0, m, 224, P - 448, 0)
    sz1 = R + 912
    d2 = _aggs(d1, 224, m, 448, sz1, 0)

    w1 = w1_ref[...]
    dot = lambda u, v: jnp.dot(u, v, preferred_element_type=jnp.float32)
    z1 = (dot(xpad[448:448 + sz1], w1[0:C])
          + dot(d1[224:224 + sz1], w1[C:2 * C])
          + dot(d2, w1[2 * C:3 * C]))

    sz2 = R + 464
    y1 = (_aggs(z1, 448, m, 672, sz2, 4)
          + m[672:672 + sz2, 8:9] * z1[224:224 + sz2] + b1_ref[...])

    mu = jnp.mean(y1, axis=-1, keepdims=True)
    var = jnp.mean((y1 - mu) ** 2, axis=-1, keepdims=True)
    y = (y1 - mu) * jax.lax.rsqrt(var + 1e-5) * gam_ref[...] + bet_ref[...]
    y = jnp.maximum(y, 0.0)

    z2 = dot(y, w2_ref[...])
    y2 = (_aggs(z2, 672, m, HALO, R, 4)
          + m[HALO:HALO + R, 8:9] * z2[232:232 + R] + b2_ref[...])

    xc = xc_ref[...]
    g = jnp.clip(gate_ref[0, 0], 0.0, 1.0)
    o_ref[...] = xc + g * (y2 - xc)


def kernel(feat, rgt_map, W1, b1, W2, b2, gamma, beta, gate):
    r2d = rgt_map.reshape(H, W)
    maps3 = pl.pallas_call(
        _maps_kernel,
        out_shape=jax.ShapeDtypeStruct((NM, H, W), jnp.float32),
    )(r2d)
    maps = jnp.transpose(maps3.reshape(NM, N), (1, 0))

    x2d = jnp.transpose(feat, (0, 2, 3, 1)).reshape(N, C)

    xspec = lambda f: pl.BlockSpec((R, C), lambda i, f=f: (f(i), 0))
    mspec = lambda f: pl.BlockSpec((R, NM), lambda i, f=f: (f(i), 0))
    prev = lambda i: jnp.maximum(i - 1, 0)
    nxt = lambda i: jnp.minimum(i + 1, NT - 1)
    cur = lambda i: i
    full = lambda r, c: pl.BlockSpec((r, c), lambda i: (0, 0))

    out_rows = pl.pallas_call(
        _main_kernel,
        grid=(NT,),
        in_specs=[
            xspec(prev), xspec(cur), xspec(nxt),
            mspec(prev), mspec(cur), mspec(nxt),
            full(3 * C, C), full(1, C), full(C, C), full(1, C),
            full(1, C), full(1, C), full(1, 1),
        ],
        out_specs=pl.BlockSpec((R, C), lambda i: (i, 0)),
        out_shape=jax.ShapeDtypeStruct((N, C), jnp.float32),
    )(x2d, x2d, x2d, maps, maps, maps,
      W1, b1.reshape(1, C), W2, b2.reshape(1, C),
      gamma.reshape(1, C), beta.reshape(1, C),
      jnp.asarray(gate, jnp.float32).reshape(1, 1))

    return jnp.transpose(out_rows.reshape(1, H, W, C), (0, 3, 1, 2))


# speedup vs baseline: 1.1693x; 1.0074x over previous
import jax
import jax.numpy as jnp
from jax.experimental import pallas as pl

SIGMA = 0.08
H = 224
W = 224
N = H * W
C = 128
R = 3584
NT = N // R
HB = 512
HALO = 2 * HB
P = R + 2 * HALO
E = HALO - 904
NM = 9


def _sh2(a, dx, dy):
    if dx == 1:
        a = jnp.concatenate([a[1:, :], jnp.zeros((1, W), a.dtype)], axis=0)
    elif dx == -1:
        a = jnp.concatenate([jnp.zeros((1, W), a.dtype), a[:-1, :]], axis=0)
    if dy == 1:
        a = jnp.concatenate([a[:, 1:], jnp.zeros((H, 1), a.dtype)], axis=1)
    elif dy == -1:
        a = jnp.concatenate([jnp.zeros((H, 1), a.dtype), a[:, :-1]], axis=1)
    return a


def _maps_kernel(r_ref, o_ref):
    r = r_ref[...]
    rmin = jnp.min(r)
    rmax = jnp.max(r)
    r01 = (r - rmin) / jnp.maximum(rmax - rmin, 1e-12)
    inv = 1.0 / (2.0 * SIGMA * SIGMA + 1e-12)
    dv = r01[1:, :] - r01[:-1, :]
    wv = jnp.exp(-(dv * dv) * inv)
    zrow = jnp.zeros((1, W), jnp.float32)
    wdn = jnp.concatenate([wv, zrow], axis=0)
    wup = jnp.concatenate([zrow, wv], axis=0)
    dh = r01[:, 1:] - r01[:, :-1]
    wh = jnp.exp(-(dh * dh) * inv)
    zcol = jnp.zeros((H, 1), jnp.float32)
    wrt = jnp.concatenate([wh, zcol], axis=1)
    wlf = jnp.concatenate([zcol, wh], axis=1)
    deg = wup + wdn + wlf + wrt
    s = (deg + 1e-12) ** -0.5
    s2 = (deg + 1.0) ** -0.5
    o_ref[0] = wup * s * _sh2(s, -1, 0)
    o_ref[1] = wdn * s * _sh2(s, 1, 0)
    o_ref[2] = wlf * s * _sh2(s, 0, -1)
    o_ref[3] = wrt * s * _sh2(s, 0, 1)
    o_ref[4] = wup * s2 * _sh2(s2, -1, 0)
    o_ref[5] = wdn * s2 * _sh2(s2, 1, 0)
    o_ref[6] = wlf * s2 * _sh2(s2, 0, -1)
    o_ref[7] = wrt * s2 * _sh2(s2, 0, 1)
    o_ref[8] = s2 * s2


def _aggs(q, qoff, m, pa, sz, c0):
    o = pa - qoff
    wu = m[pa:pa + sz, c0:c0 + 1]
    wd = m[pa:pa + sz, c0 + 1:c0 + 2]
    wl = m[pa:pa + sz, c0 + 2:c0 + 3]
    wr = m[pa:pa + sz, c0 + 3:c0 + 4]
    return (wu * q[o - W:o - W + sz] + wd * q[o + W:o + W + sz]
            + wl * q[o - 1:o - 1 + sz] + wr * q[o + 1:o + 1 + sz])


def _main_kernel(xp2_ref, xp1_ref, xc_ref, xn1_ref, xn2_ref,
                 mp2_ref, mp1_ref, mc_ref, mn1_ref, mn2_ref,
                 w1_ref, b1_ref, w2_ref, b2_ref, gam_ref, bet_ref, gate_ref,
                 o_ref):
    xpad = jnp.concatenate(
        [xp2_ref[...], xp1_ref[...], xc_ref[...], xn1_ref[...], xn2_ref[...]],
        axis=0)
    m = jnp.concatenate(
        [mp2_ref[...], mp1_ref[...], mc_ref[...], mn1_ref[...], mn2_ref[...]],
        axis=0)

    d1 = _aggs(xpad, 0, m, E + 224, P - 2 * E - 448, 0)
    sz1 = R + 912
    d2 = _aggs(d1, E + 224, m, E + 448, sz1, 0)

    w1 = w1_ref[...]
    dot = lambda u, v: jnp.dot(u, v, preferred_element_type=jnp.float32)
    a1 = E + 448
    z1 = (dot(xpad[a1:a1 + sz1], w1[0:C])
          + dot(d1[224:224 + sz1], w1[C:2 * C])
          + dot(d2, w1[2 * C:3 * C]))

    sz2 = R + 464
    a2 = E + 672
    y1 = (_aggs(z1, a1, m, a2, sz2, 4)
          + m[a2:a2 + sz2, 8:9] * z1[224:224 + sz2] + b1_ref[...])

    mu = jnp.mean(y1, axis=-1, keepdims=True)
    var = jnp.mean((y1 - mu) ** 2, axis=-1, keepdims=True)
    y = (y1 - mu) * jax.lax.rsqrt(var + 1e-5) * gam_ref[...] + bet_ref[...]
    y = jnp.maximum(y, 0.0)

    z2 = dot(y, w2_ref[...])
    y2 = (_aggs(z2, a2, m, HALO, R, 4)
          + m[HALO:HALO + R, 8:9] * z2[232:232 + R] + b2_ref[...])

    xc = xc_ref[...]
    g = jnp.clip(gate_ref[0, 0], 0.0, 1.0)
    o_ref[...] = xc + g * (y2 - xc)


def kernel(feat, rgt_map, W1, b1, W2, b2, gamma, beta, gate):
    r2d = rgt_map.reshape(H, W)
    maps3 = pl.pallas_call(
        _maps_kernel,
        out_shape=jax.ShapeDtypeStruct((NM, H, W), jnp.float32),
    )(r2d)
    maps = jnp.transpose(maps3.reshape(NM, N), (1, 0))

    x2d = jnp.transpose(feat, (0, 2, 3, 1)).reshape(N, C)

    nhb = N // HB - 1
    side = lambda d: lambda i, d=d: jnp.clip(7 * i + d, 0, nhb)
    xside = lambda d: pl.BlockSpec((HB, C), lambda i, d=d: (side(d)(i), 0))
    mside = lambda d: pl.BlockSpec((HB, NM), lambda i, d=d: (side(d)(i), 0))
    full = lambda r, c: pl.BlockSpec((r, c), lambda i: (0, 0))

    out_rows = pl.pallas_call(
        _main_kernel,
        grid=(NT,),
        in_specs=[
            xside(-2), xside(-1), pl.BlockSpec((R, C), lambda i: (i, 0)),
            xside(7), xside(8),
            mside(-2), mside(-1), pl.BlockSpec((R, NM), lambda i: (i, 0)),
            mside(7), mside(8),
            full(3 * C, C), full(1, C), full(C, C), full(1, C),
            full(1, C), full(1, C), full(1, 1),
        ],
        out_specs=pl.BlockSpec((R, C), lambda i: (i, 0)),
        out_shape=jax.ShapeDtypeStruct((N, C), jnp.float32),
    )(x2d, x2d, x2d, x2d, x2d, maps, maps, maps, maps, maps,
      W1, b1.reshape(1, C), W2, b2.reshape(1, C),
      gamma.reshape(1, C), beta.reshape(1, C),
      jnp.asarray(gate, jnp.float32).reshape(1, 1))

    return jnp.transpose(out_rows.reshape(1, H, W, C), (0, 3, 1, 2))


# speedup vs baseline: 1.4061x; 1.2025x over previous
import jax
import jax.numpy as jnp
from jax.experimental import pallas as pl

SIGMA = 0.08
H = 224
W = 224
N = H * W
C = 128
R = 3584
NT = N // R
HB = 512
HALO = 2 * HB
P = R + 2 * HALO
E = HALO - 904
NM = 9


def _sh2(a, dx, dy):
    if dx == 1:
        a = jnp.concatenate([a[1:, :], jnp.zeros((1, W), a.dtype)], axis=0)
    elif dx == -1:
        a = jnp.concatenate([jnp.zeros((1, W), a.dtype), a[:-1, :]], axis=0)
    if dy == 1:
        a = jnp.concatenate([a[:, 1:], jnp.zeros((H, 1), a.dtype)], axis=1)
    elif dy == -1:
        a = jnp.concatenate([jnp.zeros((H, 1), a.dtype), a[:, :-1]], axis=1)
    return a


def _maps_kernel(r_ref, o_ref):
    r = r_ref[...]
    rmin = jnp.min(r)
    rmax = jnp.max(r)
    r01 = (r - rmin) / jnp.maximum(rmax - rmin, 1e-12)
    inv = 1.0 / (2.0 * SIGMA * SIGMA + 1e-12)
    dv = r01[1:, :] - r01[:-1, :]
    wv = jnp.exp(-(dv * dv) * inv)
    zrow = jnp.zeros((1, W), jnp.float32)
    wdn = jnp.concatenate([wv, zrow], axis=0)
    wup = jnp.concatenate([zrow, wv], axis=0)
    dh = r01[:, 1:] - r01[:, :-1]
    wh = jnp.exp(-(dh * dh) * inv)
    zcol = jnp.zeros((H, 1), jnp.float32)
    wrt = jnp.concatenate([wh, zcol], axis=1)
    wlf = jnp.concatenate([zcol, wh], axis=1)
    deg = wup + wdn + wlf + wrt
    s = (deg + 1e-12) ** -0.5
    s2 = (deg + 1.0) ** -0.5
    o_ref[0] = wup * s * _sh2(s, -1, 0)
    o_ref[1] = wdn * s * _sh2(s, 1, 0)
    o_ref[2] = wlf * s * _sh2(s, 0, -1)
    o_ref[3] = wrt * s * _sh2(s, 0, 1)
    o_ref[4] = wup * s2 * _sh2(s2, -1, 0)
    o_ref[5] = wdn * s2 * _sh2(s2, 1, 0)
    o_ref[6] = wlf * s2 * _sh2(s2, 0, -1)
    o_ref[7] = wrt * s2 * _sh2(s2, 0, 1)
    o_ref[8] = s2 * s2


def _aggs(q, qoff, m, pa, sz, c0):
    o = pa - qoff
    wu = m[pa:pa + sz, c0:c0 + 1]
    wd = m[pa:pa + sz, c0 + 1:c0 + 2]
    wl = m[pa:pa + sz, c0 + 2:c0 + 3]
    wr = m[pa:pa + sz, c0 + 3:c0 + 4]
    return (wu * q[o - W:o - W + sz] + wd * q[o + W:o + W + sz]
            + wl * q[o - 1:o - 1 + sz] + wr * q[o + 1:o + 1 + sz])


def _main_kernel(xp2_ref, xp1_ref, xc_ref, xn1_ref, xn2_ref,
                 mp2_ref, mp1_ref, mc_ref, mn1_ref, mn2_ref,
                 w1_ref, b1_ref, w2_ref, b2_ref, gam_ref, bet_ref, gate_ref,
                 o_ref):
    bf = jnp.bfloat16
    xpad = jnp.concatenate(
        [xp2_ref[...].astype(bf), xp1_ref[...].astype(bf),
         xc_ref[...].astype(bf), xn1_ref[...].astype(bf),
         xn2_ref[...].astype(bf)], axis=0)
    m = jnp.concatenate(
        [mp2_ref[...], mp1_ref[...], mc_ref[...], mn1_ref[...], mn2_ref[...]],
        axis=0)
    mb = m[:, 0:4].astype(bf)

    d1 = _aggs(xpad, 0, mb, E + 224, P - 2 * E - 448, 0)
    sz1 = R + 912
    d2 = _aggs(d1, E + 224, mb, E + 448, sz1, 0)

    w1 = w1_ref[...].astype(bf)
    dot = lambda u, v: jnp.dot(u, v, preferred_element_type=jnp.float32)
    a1 = E + 448
    z1 = (dot(xpad[a1:a1 + sz1], w1[0:C])
          + dot(d1[224:224 + sz1], w1[C:2 * C])
          + dot(d2, w1[2 * C:3 * C]))

    sz2 = R + 464
    a2 = E + 672
    y1 = (_aggs(z1, a1, m, a2, sz2, 4)
          + m[a2:a2 + sz2, 8:9] * z1[224:224 + sz2] + b1_ref[...])

    mu = jnp.mean(y1, axis=-1, keepdims=True)
    var = jnp.mean((y1 - mu) ** 2, axis=-1, keepdims=True)
    y = (y1 - mu) * jax.lax.rsqrt(var + 1e-5) * gam_ref[...] + bet_ref[...]
    y = jnp.maximum(y, 0.0).astype(bf)

    z2 = dot(y, w2_ref[...].astype(bf))
    y2 = (_aggs(z2, a2, m, HALO, R, 4)
          + m[HALO:HALO + R, 8:9] * z2[232:232 + R] + b2_ref[...])

    xc = xc_ref[...]
    g = jnp.clip(gate_ref[0, 0], 0.0, 1.0)
    o_ref[...] = xc + g * (y2 - xc)


def kernel(feat, rgt_map, W1, b1, W2, b2, gamma, beta, gate):
    r2d = rgt_map.reshape(H, W)
    maps3 = pl.pallas_call(
        _maps_kernel,
        out_shape=jax.ShapeDtypeStruct((NM, H, W), jnp.float32),
    )(r2d)
    maps = jnp.transpose(maps3.reshape(NM, N), (1, 0))

    x2d = jnp.transpose(feat, (0, 2, 3, 1)).reshape(N, C)

    nhb = N // HB - 1
    side = lambda d: lambda i, d=d: jnp.clip(7 * i + d, 0, nhb)
    xside = lambda d: pl.BlockSpec((HB, C), lambda i, d=d: (side(d)(i), 0))
    mside = lambda d: pl.BlockSpec((HB, NM), lambda i, d=d: (side(d)(i), 0))
    full = lambda r, c: pl.BlockSpec((r, c), lambda i: (0, 0))

    out_rows = pl.pallas_call(
        _main_kernel,
        grid=(NT,),
        in_specs=[
            xside(-2), xside(-1), pl.BlockSpec((R, C), lambda i: (i, 0)),
            xside(7), xside(8),
            mside(-2), mside(-1), pl.BlockSpec((R, NM), lambda i: (i, 0)),
            mside(7), mside(8),
            full(3 * C, C), full(1, C), full(C, C), full(1, C),
            full(1, C), full(1, C), full(1, 1),
        ],
        out_specs=pl.BlockSpec((R, C), lambda i: (i, 0)),
        out_shape=jax.ShapeDtypeStruct((N, C), jnp.float32),
    )(x2d, x2d, x2d, x2d, x2d, maps, maps, maps, maps, maps,
      W1, b1.reshape(1, C), W2, b2.reshape(1, C),
      gamma.reshape(1, C), beta.reshape(1, C),
      jnp.asarray(gate, jnp.float32).reshape(1, 1))

    return jnp.transpose(out_rows.reshape(1, H, W, C), (0, 3, 1, 2))


# speedup vs baseline: 1.5391x; 1.0946x over previous
import jax
import jax.numpy as jnp
from jax.experimental import pallas as pl

SIGMA = 0.08
H = 224
W = 224
N = H * W
C = 128
R = 3584
NT = N // R
HB = 512
HALO = 2 * HB
P = R + 2 * HALO
E = HALO - 904
NM = 9


def _sh2(a, dx, dy):
    if dx == 1:
        a = jnp.concatenate([a[1:, :], jnp.zeros((1, W), a.dtype)], axis=0)
    elif dx == -1:
        a = jnp.concatenate([jnp.zeros((1, W), a.dtype), a[:-1, :]], axis=0)
    if dy == 1:
        a = jnp.concatenate([a[:, 1:], jnp.zeros((H, 1), a.dtype)], axis=1)
    elif dy == -1:
        a = jnp.concatenate([jnp.zeros((H, 1), a.dtype), a[:, :-1]], axis=1)
    return a


def _maps_kernel(r_ref, o_ref):
    r = r_ref[...]
    rmin = jnp.min(r)
    rmax = jnp.max(r)
    r01 = (r - rmin) / jnp.maximum(rmax - rmin, 1e-12)
    inv = 1.0 / (2.0 * SIGMA * SIGMA + 1e-12)
    dv = r01[1:, :] - r01[:-1, :]
    wv = jnp.exp(-(dv * dv) * inv)
    zrow = jnp.zeros((1, W), jnp.float32)
    wdn = jnp.concatenate([wv, zrow], axis=0)
    wup = jnp.concatenate([zrow, wv], axis=0)
    dh = r01[:, 1:] - r01[:, :-1]
    wh = jnp.exp(-(dh * dh) * inv)
    zcol = jnp.zeros((H, 1), jnp.float32)
    wrt = jnp.concatenate([wh, zcol], axis=1)
    wlf = jnp.concatenate([zcol, wh], axis=1)
    deg = wup + wdn + wlf + wrt
    s = (deg + 1e-12) ** -0.5
    s2 = (deg + 1.0) ** -0.5
    o_ref[0] = wup * s * _sh2(s, -1, 0)
    o_ref[1] = wdn * s * _sh2(s, 1, 0)
    o_ref[2] = wlf * s * _sh2(s, 0, -1)
    o_ref[3] = wrt * s * _sh2(s, 0, 1)
    o_ref[4] = wup * s2 * _sh2(s2, -1, 0)
    o_ref[5] = wdn * s2 * _sh2(s2, 1, 0)
    o_ref[6] = wlf * s2 * _sh2(s2, 0, -1)
    o_ref[7] = wrt * s2 * _sh2(s2, 0, 1)
    o_ref[8] = s2 * s2


def _aggs(q, qoff, m, pa, sz, c0):
    o = pa - qoff
    wu = m[pa:pa + sz, c0:c0 + 1]
    wd = m[pa:pa + sz, c0 + 1:c0 + 2]
    wl = m[pa:pa + sz, c0 + 2:c0 + 3]
    wr = m[pa:pa + sz, c0 + 3:c0 + 4]
    return (wu * q[o - W:o - W + sz] + wd * q[o + W:o + W + sz]
            + wl * q[o - 1:o - 1 + sz] + wr * q[o + 1:o + 1 + sz])


def _main_kernel(xp2_ref, xp1_ref, xc_ref, xn1_ref, xn2_ref,
                 mp2_ref, mp1_ref, mc_ref, mn1_ref, mn2_ref,
                 w1_ref, b1_ref, w2_ref, b2_ref, gam_ref, bet_ref, gate_ref,
                 o_ref):
    bf = jnp.bfloat16
    xpad = jnp.concatenate(
        [xp2_ref[...].astype(bf), xp1_ref[...].astype(bf),
         xc_ref[...].astype(bf), xn1_ref[...].astype(bf),
         xn2_ref[...].astype(bf)], axis=0)
    m = jnp.concatenate(
        [mp2_ref[...], mp1_ref[...], mc_ref[...], mn1_ref[...], mn2_ref[...]],
        axis=0)
    mb = m[:, 0:4].astype(bf)
    gb = m[:, 4:8].astype(bf)

    d1 = _aggs(xpad, 0, mb, E + 224, P - 2 * E - 448, 0)
    sz1 = R + 912
    d2 = _aggs(d1, E + 224, mb, E + 448, sz1, 0)

    w1 = w1_ref[...].astype(bf)
    dot = lambda u, v: jnp.dot(u, v, preferred_element_type=jnp.float32)
    a1 = E + 448
    z1 = (dot(xpad[a1:a1 + sz1], w1[0:C])
          + dot(d1[224:224 + sz1], w1[C:2 * C])
          + dot(d2, w1[2 * C:3 * C]))

    sz2 = R + 464
    a2 = E + 672
    z1b = z1.astype(bf)
    y1 = (_aggs(z1b, a1, gb, a2, sz2, 0)
          + m[a2:a2 + sz2, 8:9] * z1[224:224 + sz2] + b1_ref[...])

    mu = jnp.mean(y1, axis=-1, keepdims=True)
    var = jnp.mean((y1 - mu) ** 2, axis=-1, keepdims=True)
    y = (y1 - mu) * jax.lax.rsqrt(var + 1e-5) * gam_ref[...] + bet_ref[...]
    y = jnp.maximum(y, 0.0).astype(bf)

    z2 = dot(y, w2_ref[...].astype(bf))
    z2b = z2.astype(bf)
    y2 = (_aggs(z2b, a2, gb, HALO, R, 0)
          + m[HALO:HALO + R, 8:9] * z2[232:232 + R] + b2_ref[...])

    xc = xc_ref[...]
    g = jnp.clip(gate_ref[0, 0], 0.0, 1.0)
    o_ref[...] = xc + g * (y2 - xc)


def kernel(feat, rgt_map, W1, b1, W2, b2, gamma, beta, gate):
    r2d = rgt_map.reshape(H, W)
    maps3 = pl.pallas_call(
        _maps_kernel,
        out_shape=jax.ShapeDtypeStruct((NM, H, W), jnp.float32),
    )(r2d)
    maps = jnp.transpose(maps3.reshape(NM, N), (1, 0))

    x2d = jnp.transpose(feat, (0, 2, 3, 1)).reshape(N, C)

    nhb = N // HB - 1
    side = lambda d: lambda i, d=d: jnp.clip(7 * i + d, 0, nhb)
    xside = lambda d: pl.BlockSpec((HB, C), lambda i, d=d: (side(d)(i), 0))
    mside = lambda d: pl.BlockSpec((HB, NM), lambda i, d=d: (side(d)(i), 0))
    full = lambda r, c: pl.BlockSpec((r, c), lambda i: (0, 0))

    out_rows = pl.pallas_call(
        _main_kernel,
        grid=(NT,),
        in_specs=[
            xside(-2), xside(-1), pl.BlockSpec((R, C), lambda i: (i, 0)),
            xside(7), xside(8),
            mside(-2), mside(-1), pl.BlockSpec((R, NM), lambda i: (i, 0)),
            mside(7), mside(8),
            full(3 * C, C), full(1, C), full(C, C), full(1, C),
            full(1, C), full(1, C), full(1, 1),
        ],
        out_specs=pl.BlockSpec((R, C), lambda i: (i, 0)),
        out_shape=jax.ShapeDtypeStruct((N, C), jnp.float32),
    )(x2d, x2d, x2d, x2d, x2d, maps, maps, maps, maps, maps,
      W1, b1.reshape(1, C), W2, b2.reshape(1, C),
      gamma.reshape(1, C), beta.reshape(1, C),
      jnp.asarray(gate, jnp.float32).reshape(1, 1))

    return jnp.transpose(out_rows.reshape(1, H, W, C), (0, 3, 1, 2))
